# baseline (speedup 1.0000x reference)
import functools

import jax
import jax.numpy as jnp
from jax import lax
from jax.experimental import pallas as pl
from jax.experimental.pallas import tpu as pltpu
from jax.experimental.pallas import tpu_sc as plsc

H = 128
DEPTH = 5
NC = 2
NS = 16
NW = NC * NS
BF = jnp.bfloat16
F32 = jnp.float32
HW = H // 2
_MASK_HI = -65536
_RND = 32768
_UNTILED = pltpu.CompilerParams(use_tc_tiling_on_sc=False)


def _bf_unpack(w):
    lo = lax.bitcast_convert_type(lax.shift_left(w, 16), F32)
    hi = lax.bitcast_convert_type(jnp.bitwise_and(w, _MASK_HI), F32)
    return lo, hi


def _bf_pack(lo, hi):
    li = lax.shift_right_logical(
        lax.bitcast_convert_type(lo, jnp.int32) + _RND, 16)
    hh = jnp.bitwise_and(
        lax.bitcast_convert_type(hi, jnp.int32) + _RND, _MASK_HI)
    return jnp.bitwise_or(li, hh)


def _pack_tc(x):
    return lax.bitcast_convert_type(
        x.astype(BF).reshape(x.shape[0], HW, 2), jnp.int32)


def _unpack_tc(x32):
    return lax.bitcast_convert_type(x32, BF).reshape(
        x32.shape[0], H).astype(F32)


def _mm_body(x_ref, w_ref, o_ref):
    o_ref[...] = jnp.dot(x_ref[...], w_ref[...],
                         preferred_element_type=jnp.float32)


def _mmbf_body(x_ref, w_ref, o_ref):
    m = x_ref[...].astype(F32)
    o_ref[...] = jnp.dot(m, w_ref[...],
                         preferred_element_type=F32).astype(BF)


def _matmul_bf(x_bf, w, blk):
    m = x_bf.shape[0]
    n = w.shape[1]
    return pl.pallas_call(
        _mmbf_body,
        grid=(m // blk,),
        in_specs=[
            pl.BlockSpec((blk, n), lambda i: (i, 0)),
            pl.BlockSpec((n, n), lambda i: (0, 0)),
        ],
        out_specs=pl.BlockSpec((blk, n), lambda i: (i, 0)),
        out_shape=jax.ShapeDtypeStruct((m, n), BF),
    )(x_bf, w)


def _matmul(x, w, blk):
    m, k = x.shape
    n = w.shape[1]
    return pl.pallas_call(
        _mm_body,
        grid=(m // blk,),
        in_specs=[
            pl.BlockSpec((blk, k), lambda i: (i, 0)),
            pl.BlockSpec((k, n), lambda i: (0, 0)),
        ],
        out_specs=pl.BlockSpec((blk, n), lambda i: (i, 0)),
        out_shape=jax.ShapeDtypeStruct((m, n), jnp.float32),
    )(x, w)


def _mm_relu_body(x_ref, w_ref, inp_ref, msg_ref):
    acc = jnp.dot(x_ref[...], w_ref[...], preferred_element_type=jnp.float32)
    inp_ref[...] = acc
    msg_ref[...] = jnp.maximum(acc, 0.0).astype(BF)


def _input_matmul(f_bonds, W_i, blk):
    m, k = f_bonds.shape
    n = W_i.shape[1]
    return pl.pallas_call(
        _mm_relu_body,
        grid=(m // blk,),
        in_specs=[
            pl.BlockSpec((blk, k), lambda i: (i, 0)),
            pl.BlockSpec((k, n), lambda i: (0, 0)),
        ],
        out_specs=[
            pl.BlockSpec((blk, n), lambda i: (i, 0)),
            pl.BlockSpec((blk, n), lambda i: (i, 0)),
        ],
        out_shape=[jax.ShapeDtypeStruct((m, n), jnp.float32),
                   jax.ShapeDtypeStruct((m, n), BF)],
    )(f_bonds, W_i)


MOLP = 512


def _readout_body(fa_ref, am_ref, ids_ref, wo1_ref, wo2_ref, bo_ref,
                  out_ref, cnt_ref):
    i = pl.program_id(0)
    hidden = jnp.maximum(
        jnp.dot(fa_ref[...], wo1_ref[...], preferred_element_type=jnp.float32)
        + jnp.dot(am_ref[...].astype(F32), wo2_ref[...],
                  preferred_element_type=jnp.float32)
        + bo_ref[...],
        0.0,
    )
    ids = ids_ref[0, 0, :]
    onehot = (lax.broadcasted_iota(jnp.int32, (MOLP, ids.shape[0]), 0)
              == ids[None, :]).astype(jnp.float32)
    part = jnp.dot(onehot, hidden, preferred_element_type=jnp.float32)
    cpart = jnp.sum(onehot, axis=1, keepdims=True)

    @pl.when(i == 0)
    def _():
        out_ref[...] = jnp.zeros_like(out_ref)
        cnt_ref[...] = jnp.zeros_like(cnt_ref)

    out_ref[...] += part
    cnt_ref[...] += jnp.broadcast_to(cpart, cnt_ref.shape)

    @pl.when(i == pl.num_programs(0) - 1)
    def _():
        out_ref[...] = out_ref[...] / jnp.maximum(cnt_ref[...], 1.0)


def _readout(f_atoms, a_msg, ids3, Wo1, Wo2, b_o, blk):
    na, fa = f_atoms.shape
    grid = na // blk
    return pl.pallas_call(
        _readout_body,
        grid=(grid,),
        in_specs=[
            pl.BlockSpec((blk, fa), lambda i: (i, 0)),
            pl.BlockSpec((blk, H), lambda i: (i, 0)),
            pl.BlockSpec((1, 1, blk), lambda i: (i, 0, 0)),
            pl.BlockSpec((fa, H), lambda i: (0, 0)),
            pl.BlockSpec((H, H), lambda i: (0, 0)),
            pl.BlockSpec((1, H), lambda i: (0, 0)),
        ],
        out_specs=pl.BlockSpec((MOLP, H), lambda i: (0, 0)),
        out_shape=jax.ShapeDtypeStruct((MOLP, H), jnp.float32),
        scratch_shapes=[pltpu.VMEM((MOLP, H), jnp.float32)],
    )(f_atoms, a_msg, ids3, Wo1, Wo2, b_o)


_MESH = plsc.VectorSubcoreMesh(core_axis_name="c", subcore_axis_name="s")

NAP = 10240
APS = NAP // NC
APT = APS // NS
GROWS = 128
GCHUNKS = APT * 32 // GROWS


def _sum_chunk(rows, k, acc_v):
    for a in range(4):
        for g in range(HW // 16):
            sl = pl.ds(g * 16, 16)
            pairs = [_bf_unpack(rows[a * 32 + r, sl]) for r in range(32)]
            los = [p[0] for p in pairs]
            his = [p[1] for p in pairs]
            while len(los) > 1:
                los = [los[i] + los[i + 1] for i in range(0, len(los), 2)]
                his = [his[i] + his[i + 1] for i in range(0, len(his), 2)]
            acc_v[k * 4 + a, sl] = _bf_pack(los[0], his[0])


def _gather_sum_sc(msg_hbm, a2b_hbm, out_hbm,
                   idx_v, rows_a, rows_b, acc_v, sem_a, sem_b):
    c = lax.axis_index("c")
    s = lax.axis_index("s")
    pltpu.sync_copy(
        a2b_hbm.at[pl.ds(c * (APS * 32 // GROWS) + s * GCHUNKS, GCHUNKS)],
        idx_v)
    pltpu.async_copy(msg_hbm.at[idx_v.at[0]], rows_a, sem_a)
    pltpu.async_copy(msg_hbm.at[idx_v.at[1]], rows_b, sem_b)

    def body(j, _):
        k0 = 2 * j
        pltpu.make_async_copy(msg_hbm.at[idx_v.at[k0]], rows_a, sem_a).wait()
        _sum_chunk(rows_a, k0, acc_v)

        @pl.when(k0 + 2 < GCHUNKS)
        def _():
            pltpu.async_copy(msg_hbm.at[idx_v.at[k0 + 2]], rows_a, sem_a)

        pltpu.make_async_copy(msg_hbm.at[idx_v.at[k0 + 1]], rows_b,
                              sem_b).wait()
        _sum_chunk(rows_b, k0 + 1, acc_v)

        @pl.when(k0 + 3 < GCHUNKS)
        def _():
            pltpu.async_copy(msg_hbm.at[idx_v.at[k0 + 3]], rows_b, sem_b)

        return 0

    lax.fori_loop(0, GCHUNKS // 2, body, 0)
    pltpu.sync_copy(acc_v, out_hbm.at[pl.ds(c * APS + s * APT, APT)])


@functools.partial(
    pl.kernel,
    out_type=jax.ShapeDtypeStruct((NAP, HW), jnp.int32),
    mesh=_MESH,
    scratch_types=[
        pltpu.VMEM((GCHUNKS, GROWS), jnp.int32),
        pltpu.VMEM((GROWS, HW), jnp.int32),
        pltpu.VMEM((GROWS, HW), jnp.int32),
        pltpu.VMEM((APT, HW), jnp.int32),
        pltpu.SemaphoreType.DMA,
        pltpu.SemaphoreType.DMA,
    ],
    compiler_params=_UNTILED,
)
def _gather_sum(msg_hbm, a2b_hbm, out_hbm,
                idx_v, rows_a, rows_b, acc_v, sem_a, sem_b):
    _gather_sum_sc(msg_hbm, a2b_hbm, out_hbm,
                   idx_v, rows_a, rows_b, acc_v, sem_a, sem_b)


CB = 80


def _combine_sc(nb, inp_hbm, am_hbm, h_hbm, b2a_hbm, brev_hbm, out_hbm,
                idxa_v, idxr_v, inp_v, am_v, h_v, out_v,
                sem_i, sem_a, sem_h):
    c = lax.axis_index("c")
    s = lax.axis_index("s")
    w = s * NC + c
    bpw = nb // NW
    nchunks = bpw // CB
    pltpu.sync_copy(b2a_hbm.at[w], idxa_v)
    pltpu.sync_copy(brev_hbm.at[w], idxr_v)

    def body(j, _):
        base = w * bpw + j * CB
        d_i = pltpu.async_copy(inp_hbm.at[pl.ds(base, CB)], inp_v, sem_i)
        d_a = pltpu.async_copy(am_hbm.at[idxa_v.at[j]], am_v, sem_a)
        d_h = pltpu.async_copy(h_hbm.at[idxr_v.at[j]], h_v, sem_h)
        d_i.wait()
        d_a.wait()
        d_h.wait()

        def row(r, _):
            for g in range(HW // 16):
                sl = pl.ds(g * 16, 16)
                alo, ahi = _bf_unpack(am_v[r, sl])
                hlo, hhi = _bf_unpack(h_v[r, sl])
                ilo = inp_v[r, pl.ds(g * 32, 16)]
                ihi = inp_v[r, pl.ds(g * 32 + 16, 16)]
                out_v[r, sl] = _bf_pack(
                    jnp.maximum(ilo + alo - hlo, 0.0),
                    jnp.maximum(ihi + ahi - hhi, 0.0))
            return 0

        lax.fori_loop(0, CB, row, 0)
        pltpu.sync_copy(out_v, out_hbm.at[pl.ds(base, CB)])
        return 0

    lax.fori_loop(0, nchunks, body, 0)


def _make_combine(nb):
    nchunks = nb // NW // CB

    @functools.partial(
        pl.kernel,
        out_type=jax.ShapeDtypeStruct((nb, HW), jnp.int32),
        mesh=_MESH,
        scratch_types=[
            pltpu.VMEM((nchunks, CB), jnp.int32),
            pltpu.VMEM((nchunks, CB), jnp.int32),
            pltpu.VMEM((CB, H), jnp.float32),
            pltpu.VMEM((CB, HW), jnp.int32),
            pltpu.VMEM((CB, HW), jnp.int32),
            pltpu.VMEM((CB, HW), jnp.int32),
            pltpu.SemaphoreType.DMA,
            pltpu.SemaphoreType.DMA,
            pltpu.SemaphoreType.DMA,
        ],
        compiler_params=_UNTILED,
    )
    def _combine(inp_hbm, am_hbm, h_hbm, b2a_hbm, brev_hbm, out_hbm,
                 idxa_v, idxr_v, inp_v, am_v, h_v, out_v,
                 sem_i, sem_a, sem_h):
        _combine_sc(nb, inp_hbm, am_hbm, h_hbm, b2a_hbm, brev_hbm, out_hbm,
                    idxa_v, idxr_v, inp_v, am_v, h_v, out_v,
                    sem_i, sem_a, sem_h)

    return _combine


def kernel(f_atoms, f_bonds, a2b, b2a, b2revb, mol_ids, W_i, W_h, W_o, b_o):
    na, fa_dim = f_atoms.shape
    nb = f_bonds.shape[0]
    maxnb = a2b.shape[1]

    a2b_p = jnp.pad(a2b.astype(jnp.int32), ((0, NAP - na), (0, 0)))
    a2b_rs = a2b_p.reshape(NAP * maxnb // GROWS, GROWS)
    nchunks = nb // NW // CB
    b2a_rs = b2a.astype(jnp.int32).reshape(NW, nchunks, CB)
    brev_rs = b2revb.astype(jnp.int32).reshape(NW, nchunks, CB)
    ids3 = mol_ids.astype(jnp.int32).reshape(5, 1, na // 5)
    Wo1 = W_o[:fa_dim]
    Wo2 = W_o[fa_dim:]
    bo2 = b_o.reshape(1, H)

    combine = _make_combine(nb)

    within = jnp.concatenate([jnp.arange(0, 32, 2), jnp.arange(1, 32, 2)])
    perm = (jnp.arange(0, H, 32)[:, None] + within[None, :]).reshape(-1)

    def to32(x_bf):
        return lax.bitcast_convert_type(
            x_bf.reshape(x_bf.shape[0], HW, 2), jnp.int32)

    def from32(x32):
        return lax.bitcast_convert_type(x32, BF).reshape(x32.shape[0], H)

    inp, msg_bf = _input_matmul(f_bonds, W_i, 1600)
    inp_re = jnp.take(inp, perm, axis=1)

    for _ in range(DEPTH - 1):
        msg32 = to32(msg_bf)
        ga32 = _gather_sum(msg32, a2b_rs)
        h_bf = _matmul_bf(msg_bf, W_h, 1600)
        am_bf = _matmul_bf(from32(ga32), W_h, 2048)
        pre32 = combine(inp_re, to32(am_bf), to32(h_bf), b2a_rs, brev_rs)
        msg_bf = from32(pre32)

    ga32 = _gather_sum(to32(msg_bf), a2b_rs)
    out = _readout(f_atoms, from32(ga32)[:na], ids3, Wo1, Wo2, bo2, 2000)
    n_mols = 500
    return out[:n_mols]


# speedup vs baseline: 2.9202x; 2.9202x over previous
import functools

import jax
import jax.numpy as jnp
from jax import lax
from jax.experimental import pallas as pl
from jax.experimental.pallas import tpu as pltpu
from jax.experimental.pallas import tpu_sc as plsc

H = 128
DEPTH = 5
NC = 2
NS = 16
NW = NC * NS


def _pack_tc(x):
    return lax.bitcast_convert_type(
        x.astype(BF).reshape(x.shape[0], HW, 2), jnp.int32)


def _unpack_tc(x32):
    return lax.bitcast_convert_type(x32, BF).reshape(
        x32.shape[0], H).astype(F32)


def _mm_body(x_ref, w_ref, o_ref):
    o_ref[...] = jnp.dot(x_ref[...], w_ref[...],
                         preferred_element_type=jnp.float32)


def _matmul(x, w, blk):
    m, k = x.shape
    n = w.shape[1]
    return pl.pallas_call(
        _mm_body,
        grid=(m // blk,),
        in_specs=[
            pl.BlockSpec((blk, k), lambda i: (i, 0)),
            pl.BlockSpec((k, n), lambda i: (0, 0)),
        ],
        out_specs=pl.BlockSpec((blk, n), lambda i: (i, 0)),
        out_shape=jax.ShapeDtypeStruct((m, n), jnp.float32),
    )(x, w)


def _mm_relu_body(x_ref, w_ref, inp_ref, msg_ref):
    acc = jnp.dot(x_ref[...], w_ref[...], preferred_element_type=jnp.float32)
    inp_ref[...] = acc
    msg_ref[...] = jnp.maximum(acc, 0.0)


def _input_matmul(f_bonds, W_i, blk):
    m, k = f_bonds.shape
    n = W_i.shape[1]
    return pl.pallas_call(
        _mm_relu_body,
        grid=(m // blk,),
        in_specs=[
            pl.BlockSpec((blk, k), lambda i: (i, 0)),
            pl.BlockSpec((k, n), lambda i: (0, 0)),
        ],
        out_specs=[
            pl.BlockSpec((blk, n), lambda i: (i, 0)),
            pl.BlockSpec((blk, n), lambda i: (i, 0)),
        ],
        out_shape=[jax.ShapeDtypeStruct((m, n), jnp.float32),
                   jax.ShapeDtypeStruct((m, n), jnp.float32)],
    )(f_bonds, W_i)


MOLP = 512


def _readout_body(fa_ref, am_ref, ids_ref, wo1_ref, wo2_ref, bo_ref,
                  out_ref, cnt_ref):
    i = pl.program_id(0)
    hidden = jnp.maximum(
        jnp.dot(fa_ref[...], wo1_ref[...], preferred_element_type=jnp.float32)
        + jnp.dot(am_ref[...], wo2_ref[...],
                  preferred_element_type=jnp.float32)
        + bo_ref[...],
        0.0,
    )
    ids = ids_ref[0, 0, :]
    onehot = (lax.broadcasted_iota(jnp.int32, (MOLP, ids.shape[0]), 0)
              == ids[None, :]).astype(jnp.float32)
    part = jnp.dot(onehot, hidden, preferred_element_type=jnp.float32)
    cpart = jnp.sum(onehot, axis=1, keepdims=True)

    @pl.when(i == 0)
    def _():
        out_ref[...] = jnp.zeros_like(out_ref)
        cnt_ref[...] = jnp.zeros_like(cnt_ref)

    out_ref[...] += part
    cnt_ref[...] += jnp.broadcast_to(cpart, cnt_ref.shape)

    @pl.when(i == pl.num_programs(0) - 1)
    def _():
        out_ref[...] = out_ref[...] / jnp.maximum(cnt_ref[...], 1.0)


def _readout(f_atoms, a_msg, ids3, Wo1, Wo2, b_o, blk):
    na, fa = f_atoms.shape
    grid = na // blk
    return pl.pallas_call(
        _readout_body,
        grid=(grid,),
        in_specs=[
            pl.BlockSpec((blk, fa), lambda i: (i, 0)),
            pl.BlockSpec((blk, H), lambda i: (i, 0)),
            pl.BlockSpec((1, 1, blk), lambda i: (i, 0, 0)),
            pl.BlockSpec((fa, H), lambda i: (0, 0)),
            pl.BlockSpec((H, H), lambda i: (0, 0)),
            pl.BlockSpec((1, H), lambda i: (0, 0)),
        ],
        out_specs=pl.BlockSpec((MOLP, H), lambda i: (0, 0)),
        out_shape=jax.ShapeDtypeStruct((MOLP, H), jnp.float32),
        scratch_shapes=[pltpu.VMEM((MOLP, H), jnp.float32)],
    )(f_atoms, a_msg, ids3, Wo1, Wo2, b_o)


_MESH = plsc.VectorSubcoreMesh(core_axis_name="c", subcore_axis_name="s")

NAP = 10240
APS = NAP // NC
APT = APS // NS
GROWS = 128
GCHUNKS = APT * 32 // GROWS


def _sum_chunk(rows, k, acc_v):
    for a in range(4):
        for g in range(H // 16):
            sl = pl.ds(g * 16, 16)
            vals = [rows[a * 32 + r, sl] for r in range(32)]
            while len(vals) > 1:
                vals = [vals[i] + vals[i + 1] for i in range(0, len(vals), 2)]
            acc_v[k * 4 + a, sl] = vals[0]


def _gather_sum_sc(msg_hbm, a2b_hbm, out_hbm,
                   idx_v, rows_a, rows_b, acc_v, sem_a, sem_b):
    c = lax.axis_index("c")
    s = lax.axis_index("s")
    pltpu.sync_copy(
        a2b_hbm.at[pl.ds(c * (APS * 32 // GROWS) + s * GCHUNKS, GCHUNKS)],
        idx_v)
    pltpu.async_copy(msg_hbm.at[idx_v.at[0]], rows_a, sem_a)
    pltpu.async_copy(msg_hbm.at[idx_v.at[1]], rows_b, sem_b)

    def body(j, _):
        k0 = 2 * j
        pltpu.make_async_copy(msg_hbm.at[idx_v.at[k0]], rows_a, sem_a).wait()
        _sum_chunk(rows_a, k0, acc_v)

        @pl.when(k0 + 2 < GCHUNKS)
        def _():
            pltpu.async_copy(msg_hbm.at[idx_v.at[k0 + 2]], rows_a, sem_a)

        pltpu.make_async_copy(msg_hbm.at[idx_v.at[k0 + 1]], rows_b,
                              sem_b).wait()
        _sum_chunk(rows_b, k0 + 1, acc_v)

        @pl.when(k0 + 3 < GCHUNKS)
        def _():
            pltpu.async_copy(msg_hbm.at[idx_v.at[k0 + 3]], rows_b, sem_b)

        return 0

    lax.fori_loop(0, GCHUNKS // 2, body, 0)
    pltpu.sync_copy(acc_v, out_hbm.at[pl.ds(c * APS + s * APT, APT)])


@functools.partial(
    pl.kernel,
    out_type=jax.ShapeDtypeStruct((NAP, H), jnp.float32),
    mesh=_MESH,
    scratch_types=[
        pltpu.VMEM((GCHUNKS, GROWS), jnp.int32),
        pltpu.VMEM((GROWS, H), jnp.float32),
        pltpu.VMEM((GROWS, H), jnp.float32),
        pltpu.VMEM((APT, H), jnp.float32),
        pltpu.SemaphoreType.DMA,
        pltpu.SemaphoreType.DMA,
    ],
)
def _gather_sum(msg_hbm, a2b_hbm, out_hbm,
                idx_v, rows_a, rows_b, acc_v, sem_a, sem_b):
    _gather_sum_sc(msg_hbm, a2b_hbm, out_hbm,
                   idx_v, rows_a, rows_b, acc_v, sem_a, sem_b)


CB = 80


def _combine_sc(nb, inp_hbm, am_hbm, h_hbm, b2a_hbm, brev_hbm, out_hbm,
                idxa_v, idxr_v, inp_v, am_v, h_v, out_v,
                sem_i, sem_a, sem_h):
    c = lax.axis_index("c")
    s = lax.axis_index("s")
    w = s * NC + c
    bpw = nb // NW
    nchunks = bpw // CB
    pltpu.sync_copy(b2a_hbm.at[w], idxa_v)
    pltpu.sync_copy(brev_hbm.at[w], idxr_v)

    def body(j, _):
        base = w * bpw + j * CB
        d_i = pltpu.async_copy(inp_hbm.at[pl.ds(base, CB)], inp_v, sem_i)
        d_a = pltpu.async_copy(am_hbm.at[idxa_v.at[j]], am_v, sem_a)
        d_h = pltpu.async_copy(h_hbm.at[idxr_v.at[j]], h_v, sem_h)
        d_i.wait()
        d_a.wait()
        d_h.wait()

        def row(r, _):
            for g in range(H // 16):
                sl = pl.ds(g * 16, 16)
                out_v[r, sl] = jnp.maximum(
                    inp_v[r, sl] + am_v[r, sl] - h_v[r, sl], 0.0)
            return 0

        lax.fori_loop(0, CB, row, 0)
        pltpu.sync_copy(out_v, out_hbm.at[pl.ds(base, CB)])
        return 0

    lax.fori_loop(0, nchunks, body, 0)


def _make_combine(nb):
    nchunks = nb // NW // CB

    @functools.partial(
        pl.kernel,
        out_type=jax.ShapeDtypeStruct((nb, H), jnp.float32),
        mesh=_MESH,
        scratch_types=[
            pltpu.VMEM((nchunks, CB), jnp.int32),
            pltpu.VMEM((nchunks, CB), jnp.int32),
            pltpu.VMEM((CB, H), jnp.float32),
            pltpu.VMEM((CB, H), jnp.float32),
            pltpu.VMEM((CB, H), jnp.float32),
            pltpu.VMEM((CB, H), jnp.float32),
            pltpu.SemaphoreType.DMA,
            pltpu.SemaphoreType.DMA,
            pltpu.SemaphoreType.DMA,
        ],
    )
    def _combine(inp_hbm, am_hbm, h_hbm, b2a_hbm, brev_hbm, out_hbm,
                 idxa_v, idxr_v, inp_v, am_v, h_v, out_v,
                 sem_i, sem_a, sem_h):
        _combine_sc(nb, inp_hbm, am_hbm, h_hbm, b2a_hbm, brev_hbm, out_hbm,
                    idxa_v, idxr_v, inp_v, am_v, h_v, out_v,
                    sem_i, sem_a, sem_h)

    return _combine


def kernel(f_atoms, f_bonds, a2b, b2a, b2revb, mol_ids, W_i, W_h, W_o, b_o):
    na, fa_dim = f_atoms.shape
    nb = f_bonds.shape[0]
    maxnb = a2b.shape[1]

    a2b_p = jnp.pad(a2b.astype(jnp.int32), ((0, NAP - na), (0, 0)))
    a2b_rs = a2b_p.reshape(NAP * maxnb // GROWS, GROWS)
    nchunks = nb // NW // CB
    b2a_rs = b2a.astype(jnp.int32).reshape(NW, nchunks, CB)
    brev_rs = b2revb.astype(jnp.int32).reshape(NW, nchunks, CB)
    ids3 = mol_ids.astype(jnp.int32).reshape(5, 1, na // 5)
    Wo1 = W_o[:fa_dim]
    Wo2 = W_o[fa_dim:]
    bo2 = b_o.reshape(1, H)

    combine = _make_combine(nb)

    inp, msg = _input_matmul(f_bonds, W_i, 1600)

    for _ in range(DEPTH - 1):
        ga = _gather_sum(msg, a2b_rs)
        h = _matmul(msg, W_h, 1600)
        am_h = _matmul(ga, W_h, 2048)
        msg = combine(inp, am_h, h, b2a_rs, brev_rs)

    ga = _gather_sum(msg, a2b_rs)
    out = _readout(f_atoms, ga[:na], ids3, Wo1, Wo2, bo2, 2000)
    n_mols = 500
    return out[:n_mols]
